# baseline (speedup 1.0000x reference)
import functools

import jax
import jax.numpy as jnp
from jax import lax
from jax.experimental import pallas as pl
from jax.experimental.pallas import tpu as pltpu
from jax.experimental.pallas import tpu_sc as plsc

NC = 2
NS = 16
NW = NC * NS
B = 16384
BPW = B // NW
CHUNK = 128
NCHUNK = BPW // CHUNK
GMF_D = 32
MLP_D = 64


def _sc_gather_body(uidx_hbm, iidx_hbm, gu_t, gi_t, mu_t, mi_t,
                    gu_o, gi_o, mu_o, mi_o,
                    uidx_v, iidx_v, gu_v, gi_v, mu_v, mi_v, sem):
    wid = lax.axis_index("s") * NC + lax.axis_index("c")
    base = wid * BPW
    pltpu.sync_copy(uidx_hbm.at[wid], uidx_v)
    pltpu.sync_copy(iidx_hbm.at[wid], iidx_v)
    copies = []
    for j in range(NCHUNK):
        rows = pl.ds(j * CHUNK, CHUNK)
        copies.append(pltpu.async_copy(gu_t.at[uidx_v.at[j]], gu_v.at[rows], sem))
        copies.append(pltpu.async_copy(gi_t.at[iidx_v.at[j]], gi_v.at[rows], sem))
        copies.append(pltpu.async_copy(mu_t.at[uidx_v.at[j]], mu_v.at[rows], sem))
        copies.append(pltpu.async_copy(mi_t.at[iidx_v.at[j]], mi_v.at[rows], sem))
    for c in copies:
        c.wait()
    out_rows = pl.ds(base, BPW)
    pltpu.sync_copy(gu_v, gu_o.at[out_rows])
    pltpu.sync_copy(gi_v, gi_o.at[out_rows])
    pltpu.sync_copy(mu_v, mu_o.at[out_rows])
    pltpu.sync_copy(mi_v, mi_o.at[out_rows])


def _make_sc_gather():
    mesh = plsc.VectorSubcoreMesh(
        core_axis_name="c", subcore_axis_name="s",
        num_cores=NC, num_subcores=NS)
    return pl.kernel(
        _sc_gather_body,
        out_type=[
            jax.ShapeDtypeStruct((B, GMF_D), jnp.float32),
            jax.ShapeDtypeStruct((B, GMF_D), jnp.float32),
            jax.ShapeDtypeStruct((B, MLP_D), jnp.float32),
            jax.ShapeDtypeStruct((B, MLP_D), jnp.float32),
        ],
        mesh=mesh,
        compiler_params=pltpu.CompilerParams(use_tc_tiling_on_sc=False),
        scratch_types=[
            pltpu.VMEM((NCHUNK, CHUNK), jnp.int32),
            pltpu.VMEM((NCHUNK, CHUNK), jnp.int32),
            pltpu.VMEM((BPW, GMF_D), jnp.float32),
            pltpu.VMEM((BPW, GMF_D), jnp.float32),
            pltpu.VMEM((BPW, MLP_D), jnp.float32),
            pltpu.VMEM((BPW, MLP_D), jnp.float32),
            pltpu.SemaphoreType.DMA,
        ],
    )


BB = 2048


def _tc_mlp_body(gu, gi, mu, mi, w1u, w1i, b1, w2, b2, w3, b3,
                 gw, gb, nw0, nwr, nb, out):
    f32 = jnp.float32
    mf = gu[...] * gi[...]
    gmf_lin = jnp.sum(mf * gw[...], axis=1, keepdims=True) + gb[0, 0]
    gmf_out = jax.nn.sigmoid(gmf_lin)
    h = jnp.dot(mu[...], w1u[...], preferred_element_type=f32)
    h += jnp.dot(mi[...], w1i[...], preferred_element_type=f32)
    h = jnp.maximum(h + b1[...], 0.0)
    h = jnp.maximum(jnp.dot(h, w2[...], preferred_element_type=f32) + b2[...], 0.0)
    h = jnp.maximum(jnp.dot(h, w3[...], preferred_element_type=f32) + b3[...], 0.0)
    logit = (gmf_out * nw0[0, 0]
             + jnp.dot(h, nwr[...], preferred_element_type=f32)
             + nb[0, 0])
    out[...] = jax.nn.sigmoid(logit)


def _make_tc_mlp():
    full = lambda shape: pl.BlockSpec(shape, lambda i: (0, 0))
    row = lambda d: pl.BlockSpec((BB, d), lambda i: (i, 0))
    return pl.pallas_call(
        _tc_mlp_body,
        grid=(B // BB,),
        in_specs=[
            row(GMF_D), row(GMF_D), row(MLP_D), row(MLP_D),
            full((MLP_D, MLP_D)), full((MLP_D, MLP_D)), full((1, MLP_D)),
            full((MLP_D, 32)), full((1, 32)),
            full((32, 16)), full((1, 16)),
            full((1, GMF_D)), full((1, 1)),
            full((1, 1)), full((16, 1)), full((1, 1)),
        ],
        out_specs=pl.BlockSpec((BB, 1), lambda i: (i, 0)),
        out_shape=jax.ShapeDtypeStruct((B, 1), jnp.float32),
    )


def kernel(user_vector, item_vector, gmf_user_emb, gmf_item_emb, gmf_h_W,
           gmf_h_b, mlp_user_emb, mlp_item_emb, mlp_W1, mlp_b1, mlp_W2,
           mlp_b2, mlp_W3, mlp_b3, neumf_W, neumf_b):
    uidx = user_vector.reshape(NW, NCHUNK, CHUNK)
    iidx = item_vector.reshape(NW, NCHUNK, CHUNK)
    gu, gi, mu, mi = _make_sc_gather()(
        uidx, iidx, gmf_user_emb, gmf_item_emb, mlp_user_emb, mlp_item_emb)
    w1t = mlp_W1.T
    w1u, w1i = w1t[:MLP_D], w1t[MLP_D:]
    out = _make_tc_mlp()(
        gu, gi, mu, mi,
        w1u, w1i, mlp_b1.reshape(1, MLP_D),
        mlp_W2.T, mlp_b2.reshape(1, 32),
        mlp_W3.T, mlp_b3.reshape(1, 16),
        gmf_h_W, gmf_h_b.reshape(1, 1),
        neumf_W[:, :1], neumf_W[:, 1:].T, neumf_b.reshape(1, 1))
    return out
